# baseline (speedup 1.0000x reference)
import functools

import jax
import jax.numpy as jnp
from jax import lax
from jax.experimental import pallas as pl
from jax.experimental.pallas import tpu as pltpu
from jax.experimental.pallas import tpu_sc as plsc

N = 10000
E = 320000
DIN = 128
DH = 256
DOUT = 40
DOP = 128

NC = 2
NS = 16
NW = NC * NS

NPAD = 10240
EPAD = 327680
BLK_E = 128
NBLK = EPAD // (NW * BLK_E)
ZROWS = NPAD // NS
ZB = ZROWS // BLK_E

NBLK0 = 160
NBLK1 = 0
PNB = 16
assert NBLK0 + NBLK1 == 2 * NBLK
assert NBLK0 % (2 * PNB) == 0 and NBLK1 % (2 * PNB) == 0


def _sc_degree(dst2d):
    mesh = plsc.VectorSubcoreMesh(core_axis_name="c", subcore_axis_name="s")

    @functools.partial(
        pl.kernel,
        mesh=mesh,
        out_type=jax.ShapeDtypeStruct((NC, NPAD, 16), jnp.float32),
        compiler_params=pltpu.CompilerParams(use_tc_tiling_on_sc=False),
        scratch_types=[
            pltpu.VMEM((NBLK, BLK_E), jnp.int32),
            pltpu.VMEM((BLK_E, 16), jnp.float32),
            pltpu.VMEM((BLK_E, 16), jnp.float32),
            pltpu.VMEM_SHARED((NPAD, 16), jnp.float32),
        ],
    )
    def k(dst_hbm, out_hbm, dst_v, ones_v, zeros_v, acc):
        c = lax.axis_index("c")
        s = lax.axis_index("s")
        wid = c * NS + s

        def fill(i, carry):
            ones_v[i, :] = jnp.ones((16,), jnp.float32)
            zeros_v[i, :] = jnp.zeros((16,), jnp.float32)
            return carry

        lax.fori_loop(0, BLK_E, fill, 0)

        def zero(b, carry):
            pltpu.sync_copy(zeros_v, acc.at[pl.ds(s * ZROWS + b * BLK_E, BLK_E)])
            return carry

        lax.fori_loop(0, ZB, zero, 0)
        pltpu.sync_copy(dst_hbm.at[pl.ds(wid * NBLK, NBLK)], dst_v)
        plsc.subcore_barrier()

        def body(j, carry):
            pltpu.sync_copy(ones_v, acc.at[dst_v.at[j]], add=True)
            return carry

        lax.fori_loop(0, NBLK, body, 0)
        plsc.subcore_barrier()
        pltpu.sync_copy(
            acc.at[pl.ds(s * ZROWS, ZROWS)],
            out_hbm.at[c, pl.ds(s * ZROWS, ZROWS)],
        )

    return k(dst2d)


def _sc_agg(y, comb2d, d):
    mesh = plsc.VectorSubcoreMesh(core_axis_name="c", subcore_axis_name="s")

    @functools.partial(
        pl.kernel,
        mesh=mesh,
        out_type=jax.ShapeDtypeStruct((NC, NPAD, d), jnp.float32),
        compiler_params=pltpu.CompilerParams(use_tc_tiling_on_sc=False),
        scratch_types=[
            pltpu.VMEM((2 * PNB, BLK_E), jnp.int32),
            pltpu.VMEM((2 * PNB, BLK_E), jnp.int32),
            pltpu.VMEM((BLK_E, d), jnp.float32),
            pltpu.VMEM((BLK_E, d), jnp.float32),
            pltpu.VMEM_SHARED((NPAD, d), jnp.float32),
            pltpu.SemaphoreType.DMA,
            pltpu.SemaphoreType.DMA,
            pltpu.SemaphoreType.DMA,
            pltpu.SemaphoreType.DMA,
        ],
    )
    def k(y_hbm, comb_hbm, out_hbm, ib0, ib1, rows0, rows1, acc,
          sem0, sem1, semi0, semi1):
        c = lax.axis_index("c")
        s = lax.axis_index("s")
        base_blk = jnp.where(c == 0, s * NBLK0, NS * NBLK0 + s * NBLK1)
        nph = jnp.where(c == 0, NBLK0 // PNB, NBLK1 // PNB)

        def fetch_idx(p, buf, sem):
            pltpu.async_copy(
                comb_hbm.at[pl.ds(2 * (base_blk + p * PNB), 2 * PNB)], buf, sem)

        def drain_idx(buf, sem):
            pltpu.make_async_copy(comb_hbm.at[pl.ds(0, 2 * PNB)], buf,
                                  sem).wait()

        @pl.when(nph > 0)
        def _():
            fetch_idx(0, ib0, semi0)

        def zrow(i, carry):
            def zcol(j, carry2):
                rows0[i, pl.ds(j * 16, 16)] = jnp.zeros((16,), jnp.float32)
                return carry2

            lax.fori_loop(0, d // 16, zcol, 0)
            return carry

        lax.fori_loop(0, BLK_E, zrow, 0)

        def zero(b, carry):
            pltpu.sync_copy(rows0, acc.at[pl.ds(s * ZROWS + b * BLK_E, BLK_E)])
            return carry

        lax.fori_loop(0, ZB, zero, 0)
        plsc.subcore_barrier()

        def gather(ib, j, buf, sem):
            pltpu.async_copy(y_hbm.at[ib.at[2 * j]], buf, sem)

        def drain(buf, sem):
            pltpu.make_async_copy(y_hbm.at[pl.ds(0, BLK_E)], buf, sem).wait()

        def scatter(ib, j, buf):
            pltpu.sync_copy(buf, acc.at[ib.at[2 * j + 1]], add=True)

        def run_phase(ib):
            gather(ib, 0, rows0, sem0)

            def body(i, carry):
                j = 2 * i
                gather(ib, j + 1, rows1, sem1)
                drain(rows0, sem0)
                scatter(ib, j, rows0)
                gather(ib, j + 2, rows0, sem0)
                drain(rows1, sem1)
                scatter(ib, j + 1, rows1)
                return carry

            lax.fori_loop(0, PNB // 2 - 1, body, 0)
            gather(ib, PNB - 1, rows1, sem1)
            drain(rows0, sem0)
            scatter(ib, PNB - 2, rows0)
            drain(rows1, sem1)
            scatter(ib, PNB - 1, rows1)

        def phase_pair(p2, carry):
            p = 2 * p2

            @pl.when(p + 1 < nph)
            def _():
                fetch_idx(p + 1, ib1, semi1)

            drain_idx(ib0, semi0)
            run_phase(ib0)

            @pl.when(p + 2 < nph)
            def _():
                fetch_idx(p + 2, ib0, semi0)

            @pl.when(p + 1 < nph)
            def _():
                drain_idx(ib1, semi1)
                run_phase(ib1)

            return carry

        lax.fori_loop(0, nph // 2, phase_pair, 0)
        plsc.subcore_barrier()
        pltpu.sync_copy(
            acc.at[pl.ds(s * ZROWS, ZROWS)],
            out_hbm.at[c, pl.ds(s * ZROWS, ZROWS)],
        )

    return k(y, comb2d)


def _tc_prep(deg_parts, x_pad):
    blk = 1024

    def body(d_ref, x_ref, dinv_ref, y_ref):
        deg = d_ref[0, :, 0:1] + d_ref[1, :, 0:1] + 1.0
        dinv = lax.rsqrt(deg)
        dinv_ref[...] = dinv
        y_ref[...] = x_ref[...] * dinv

    return pl.pallas_call(
        body,
        grid=(NPAD // blk,),
        in_specs=[
            pl.BlockSpec((NC, blk, 16), lambda i: (0, i, 0)),
            pl.BlockSpec((blk, DIN), lambda i: (i, 0)),
        ],
        out_specs=[
            pl.BlockSpec((blk, 1), lambda i: (i, 0)),
            pl.BlockSpec((blk, DIN), lambda i: (i, 0)),
        ],
        out_shape=[
            jax.ShapeDtypeStruct((NPAD, 1), jnp.float32),
            jax.ShapeDtypeStruct((NPAD, DIN), jnp.float32),
        ],
    )(deg_parts, x_pad)


def _tc_mlp(agg_parts, y, dinv, w1, b1r, w2p):
    blk = 512

    def body(a_ref, y_ref, dinv_ref, w1_ref, b1_ref, w2_ref, out_ref):
        z = (a_ref[0] + a_ref[1] + y_ref[...]) * dinv_ref[...]
        h = jnp.dot(z, w1_ref[...], preferred_element_type=jnp.float32)
        h = jnp.maximum(h + b1_ref[...], 0.0)
        p = jnp.dot(h, w2_ref[...], preferred_element_type=jnp.float32)
        out_ref[...] = p * dinv_ref[...]

    return pl.pallas_call(
        body,
        grid=(NPAD // blk,),
        in_specs=[
            pl.BlockSpec((NC, blk, DIN), lambda i: (0, i, 0)),
            pl.BlockSpec((blk, DIN), lambda i: (i, 0)),
            pl.BlockSpec((blk, 1), lambda i: (i, 0)),
            pl.BlockSpec((DIN, DH), lambda i: (0, 0)),
            pl.BlockSpec((1, DH), lambda i: (0, 0)),
            pl.BlockSpec((DH, DOP), lambda i: (0, 0)),
        ],
        out_specs=pl.BlockSpec((blk, DOP), lambda i: (i, 0)),
        out_shape=jax.ShapeDtypeStruct((NPAD, DOP), jnp.float32),
    )(agg_parts, y, dinv, w1, b1r, w2p)


def _tc_logsoftmax(agg_parts, y2, dinv, b2p):
    blk = 1024

    def body(a_ref, y2_ref, dinv_ref, b2_ref, out_ref):
        z = (a_ref[0] + a_ref[1] + y2_ref[...]) * dinv_ref[...] + b2_ref[...]
        col = lax.broadcasted_iota(jnp.int32, z.shape, 1)
        mask = col < DOUT
        zm = jnp.where(mask, z, -jnp.inf)
        m = jnp.max(zm, axis=1, keepdims=True)
        e = jnp.where(mask, jnp.exp(z - m), 0.0)
        lse = jnp.log(jnp.sum(e, axis=1, keepdims=True))
        out_ref[...] = z - m - lse

    return pl.pallas_call(
        body,
        grid=(NPAD // blk,),
        in_specs=[
            pl.BlockSpec((NC, blk, DOP), lambda i: (0, i, 0)),
            pl.BlockSpec((blk, DOP), lambda i: (i, 0)),
            pl.BlockSpec((blk, 1), lambda i: (i, 0)),
            pl.BlockSpec((1, DOP), lambda i: (0, 0)),
        ],
        out_specs=pl.BlockSpec((blk, DOP), lambda i: (i, 0)),
        out_shape=jax.ShapeDtypeStruct((NPAD, DOP), jnp.float32),
    )(agg_parts, y2, dinv, b2p)


def kernel(x, edge_index, W1, b1, W2, b2):
    src = edge_index[0]
    dst = edge_index[1]
    pad = EPAD - E
    src2d = jnp.concatenate(
        [src, jnp.zeros((pad,), jnp.int32)]).reshape(EPAD // BLK_E, BLK_E)
    dst2d = jnp.concatenate(
        [dst, jnp.full((pad,), N, jnp.int32)]).reshape(EPAD // BLK_E, BLK_E)
    comb2d = jnp.stack([src2d, dst2d], axis=1).reshape(2 * EPAD // BLK_E, BLK_E)
    x_pad = jnp.pad(x, ((0, NPAD - N), (0, 0)))
    w2p = jnp.pad(W2, ((0, 0), (0, DOP - DOUT)))
    b1r = b1.reshape(1, DH)
    b2p = jnp.pad(b2, (0, DOP - DOUT)).reshape(1, DOP)

    deg_parts = _sc_degree(dst2d)
    dinv, y = _tc_prep(deg_parts, x_pad)
    agg1 = _sc_agg(y, comb2d, DIN)
    y2 = _tc_mlp(agg1, y, dinv, W1, b1r, w2p)
    agg2 = _sc_agg(y2, comb2d, DOP)
    out = _tc_logsoftmax(agg2, y2, dinv, b2p)
    return out[:N, :DOUT]


# speedup vs baseline: 1.3088x; 1.3088x over previous
import functools

import jax
import jax.numpy as jnp
from jax import lax
from jax.experimental import pallas as pl
from jax.experimental.pallas import tpu as pltpu
from jax.experimental.pallas import tpu_sc as plsc

N = 10000
E = 320000
DIN = 128
DH = 256
DOUT = 40
DOP = 128

NC = 2
NS = 16
NW = NC * NS

NPAD = 10240
EPAD = 327680
BLK_E = 128
NBLK = EPAD // (NW * BLK_E)
ZROWS = NPAD // NS
ZB = ZROWS // BLK_E

NBLK0 = 144
NBLK1 = 16
PNB = 16
assert NBLK0 + NBLK1 == 2 * NBLK
assert NBLK0 % PNB == 0 and NBLK1 % PNB == 0 and PNB % 2 == 0


def _sc_degree(dst2d):
    mesh = plsc.VectorSubcoreMesh(core_axis_name="c", subcore_axis_name="s")

    @functools.partial(
        pl.kernel,
        mesh=mesh,
        out_type=jax.ShapeDtypeStruct((NC, NPAD, 16), jnp.float32),
        compiler_params=pltpu.CompilerParams(use_tc_tiling_on_sc=False),
        scratch_types=[
            pltpu.VMEM((NBLK, BLK_E), jnp.int32),
            pltpu.VMEM((BLK_E, 16), jnp.float32),
            pltpu.VMEM((BLK_E, 16), jnp.float32),
            pltpu.VMEM_SHARED((NPAD, 16), jnp.float32),
        ],
    )
    def k(dst_hbm, out_hbm, dst_v, ones_v, zeros_v, acc):
        c = lax.axis_index("c")
        s = lax.axis_index("s")
        wid = c * NS + s

        def fill(i, carry):
            ones_v[i, :] = jnp.ones((16,), jnp.float32)
            zeros_v[i, :] = jnp.zeros((16,), jnp.float32)
            return carry

        lax.fori_loop(0, BLK_E, fill, 0)

        def zero(b, carry):
            pltpu.sync_copy(zeros_v, acc.at[pl.ds(s * ZROWS + b * BLK_E, BLK_E)])
            return carry

        lax.fori_loop(0, ZB, zero, 0)
        pltpu.sync_copy(dst_hbm.at[pl.ds(wid * NBLK, NBLK)], dst_v)
        plsc.subcore_barrier()

        def body(j, carry):
            pltpu.sync_copy(ones_v, acc.at[dst_v.at[j]], add=True)
            return carry

        lax.fori_loop(0, NBLK, body, 0)
        plsc.subcore_barrier()
        pltpu.sync_copy(
            acc.at[pl.ds(s * ZROWS, ZROWS)],
            out_hbm.at[c, pl.ds(s * ZROWS, ZROWS)],
        )

    return k(dst2d)


def _sc_agg(y, comb2d, d):
    mesh = plsc.VectorSubcoreMesh(core_axis_name="c", subcore_axis_name="s")

    @functools.partial(
        pl.kernel,
        mesh=mesh,
        out_type=jax.ShapeDtypeStruct((NC, NPAD, d), jnp.float32),
        compiler_params=pltpu.CompilerParams(use_tc_tiling_on_sc=False),
        scratch_types=[
            pltpu.VMEM((2 * PNB, BLK_E), jnp.int32),
            pltpu.VMEM((2 * PNB, BLK_E), jnp.int32),
            pltpu.VMEM((BLK_E, d), jnp.float32),
            pltpu.VMEM((BLK_E, d), jnp.float32),
            pltpu.VMEM_SHARED((NPAD, d), jnp.float32),
            pltpu.SemaphoreType.DMA,
            pltpu.SemaphoreType.DMA,
            pltpu.SemaphoreType.DMA,
            pltpu.SemaphoreType.DMA,
        ],
    )
    def k(y_hbm, comb_hbm, out_hbm, ib0, ib1, rows0, rows1, acc,
          sem0, sem1, semi0, semi1):
        c = lax.axis_index("c")
        s = lax.axis_index("s")
        base_blk = jnp.where(c == 0, s * NBLK0, NS * NBLK0 + s * NBLK1)
        nph = jnp.where(c == 0, NBLK0 // PNB, NBLK1 // PNB)

        def fetch_idx(p, buf, sem):
            pltpu.async_copy(
                comb_hbm.at[pl.ds(2 * (base_blk + p * PNB), 2 * PNB)], buf, sem)

        def drain_idx(buf, sem):
            pltpu.make_async_copy(comb_hbm.at[pl.ds(0, 2 * PNB)], buf,
                                  sem).wait()

        @pl.when(nph > 0)
        def _():
            fetch_idx(0, ib0, semi0)

        def zrow(i, carry):
            def zcol(j, carry2):
                rows0[i, pl.ds(j * 16, 16)] = jnp.zeros((16,), jnp.float32)
                return carry2

            lax.fori_loop(0, d // 16, zcol, 0)
            return carry

        lax.fori_loop(0, BLK_E, zrow, 0)

        def zero(b, carry):
            pltpu.sync_copy(rows0, acc.at[pl.ds(s * ZROWS + b * BLK_E, BLK_E)])
            return carry

        lax.fori_loop(0, ZB, zero, 0)
        plsc.subcore_barrier()

        def gather(ib, j, buf, sem):
            pltpu.async_copy(y_hbm.at[ib.at[2 * j]], buf, sem)

        def drain(buf, sem):
            pltpu.make_async_copy(y_hbm.at[pl.ds(0, BLK_E)], buf, sem).wait()

        def scatter(ib, j, buf):
            pltpu.sync_copy(buf, acc.at[ib.at[2 * j + 1]], add=True)

        def run_phase(ib):
            gather(ib, 0, rows0, sem0)

            def body(i, carry):
                j = 2 * i
                gather(ib, j + 1, rows1, sem1)
                drain(rows0, sem0)
                scatter(ib, j, rows0)
                gather(ib, j + 2, rows0, sem0)
                drain(rows1, sem1)
                scatter(ib, j + 1, rows1)
                return carry

            lax.fori_loop(0, PNB // 2 - 1, body, 0)
            gather(ib, PNB - 1, rows1, sem1)
            drain(rows0, sem0)
            scatter(ib, PNB - 2, rows0)
            drain(rows1, sem1)
            scatter(ib, PNB - 1, rows1)

        def phase(p, carry):
            @pl.when(p % 2 == 0)
            def _():
                @pl.when(p + 1 < nph)
                def _():
                    fetch_idx(p + 1, ib1, semi1)

                drain_idx(ib0, semi0)
                run_phase(ib0)

            @pl.when(p % 2 == 1)
            def _():
                @pl.when(p + 1 < nph)
                def _():
                    fetch_idx(p + 1, ib0, semi0)

                drain_idx(ib1, semi1)
                run_phase(ib1)

            return carry

        lax.fori_loop(0, nph, phase, 0)
        plsc.subcore_barrier()
        pltpu.sync_copy(
            acc.at[pl.ds(s * ZROWS, ZROWS)],
            out_hbm.at[c, pl.ds(s * ZROWS, ZROWS)],
        )

    return k(y, comb2d)


def _tc_prep(deg_parts, x_pad):
    blk = 1024

    def body(d_ref, x_ref, dinv_ref, y_ref):
        deg = d_ref[0, :, 0:1] + d_ref[1, :, 0:1] + 1.0
        dinv = lax.rsqrt(deg)
        dinv_ref[...] = dinv
        y_ref[...] = x_ref[...] * dinv

    return pl.pallas_call(
        body,
        grid=(NPAD // blk,),
        in_specs=[
            pl.BlockSpec((NC, blk, 16), lambda i: (0, i, 0)),
            pl.BlockSpec((blk, DIN), lambda i: (i, 0)),
        ],
        out_specs=[
            pl.BlockSpec((blk, 1), lambda i: (i, 0)),
            pl.BlockSpec((blk, DIN), lambda i: (i, 0)),
        ],
        out_shape=[
            jax.ShapeDtypeStruct((NPAD, 1), jnp.float32),
            jax.ShapeDtypeStruct((NPAD, DIN), jnp.float32),
        ],
    )(deg_parts, x_pad)


def _tc_mlp(agg_parts, y, dinv, w1, b1r, w2p):
    blk = 512

    def body(a_ref, y_ref, dinv_ref, w1_ref, b1_ref, w2_ref, out_ref):
        z = (a_ref[0] + a_ref[1] + y_ref[...]) * dinv_ref[...]
        h = jnp.dot(z, w1_ref[...], preferred_element_type=jnp.float32)
        h = jnp.maximum(h + b1_ref[...], 0.0)
        p = jnp.dot(h, w2_ref[...], preferred_element_type=jnp.float32)
        out_ref[...] = p * dinv_ref[...]

    return pl.pallas_call(
        body,
        grid=(NPAD // blk,),
        in_specs=[
            pl.BlockSpec((NC, blk, DIN), lambda i: (0, i, 0)),
            pl.BlockSpec((blk, DIN), lambda i: (i, 0)),
            pl.BlockSpec((blk, 1), lambda i: (i, 0)),
            pl.BlockSpec((DIN, DH), lambda i: (0, 0)),
            pl.BlockSpec((1, DH), lambda i: (0, 0)),
            pl.BlockSpec((DH, DOP), lambda i: (0, 0)),
        ],
        out_specs=pl.BlockSpec((blk, DOP), lambda i: (i, 0)),
        out_shape=jax.ShapeDtypeStruct((NPAD, DOP), jnp.float32),
    )(agg_parts, y, dinv, w1, b1r, w2p)


def _tc_logsoftmax(agg_parts, y2, dinv, b2p):
    blk = 1024

    def body(a_ref, y2_ref, dinv_ref, b2_ref, out_ref):
        z = (a_ref[0] + a_ref[1] + y2_ref[...]) * dinv_ref[...] + b2_ref[...]
        col = lax.broadcasted_iota(jnp.int32, z.shape, 1)
        mask = col < DOUT
        zm = jnp.where(mask, z, -jnp.inf)
        m = jnp.max(zm, axis=1, keepdims=True)
        e = jnp.where(mask, jnp.exp(z - m), 0.0)
        lse = jnp.log(jnp.sum(e, axis=1, keepdims=True))
        out_ref[...] = z - m - lse

    return pl.pallas_call(
        body,
        grid=(NPAD // blk,),
        in_specs=[
            pl.BlockSpec((NC, blk, DOP), lambda i: (0, i, 0)),
            pl.BlockSpec((blk, DOP), lambda i: (i, 0)),
            pl.BlockSpec((blk, 1), lambda i: (i, 0)),
            pl.BlockSpec((1, DOP), lambda i: (0, 0)),
        ],
        out_specs=pl.BlockSpec((blk, DOP), lambda i: (i, 0)),
        out_shape=jax.ShapeDtypeStruct((NPAD, DOP), jnp.float32),
    )(agg_parts, y2, dinv, b2p)


def kernel(x, edge_index, W1, b1, W2, b2):
    src = edge_index[0]
    dst = edge_index[1]
    pad = EPAD - E
    src2d = jnp.concatenate(
        [src, jnp.zeros((pad,), jnp.int32)]).reshape(EPAD // BLK_E, BLK_E)
    dst2d = jnp.concatenate(
        [dst, jnp.full((pad,), N, jnp.int32)]).reshape(EPAD // BLK_E, BLK_E)
    comb2d = jnp.stack([src2d, dst2d], axis=1).reshape(2 * EPAD // BLK_E, BLK_E)
    x_pad = jnp.pad(x, ((0, NPAD - N), (0, 0)))
    w2p = jnp.pad(W2, ((0, 0), (0, DOP - DOUT)))
    b1r = b1.reshape(1, DH)
    b2p = jnp.pad(b2, (0, DOP - DOUT)).reshape(1, DOP)

    deg_parts = _sc_degree(dst2d)
    dinv, y = _tc_prep(deg_parts, x_pad)
    agg1 = _sc_agg(y, comb2d, DIN)
    y2 = _tc_mlp(agg1, y, dinv, W1, b1r, w2p)
    agg2 = _sc_agg(y2, comb2d, DOP)
    out = _tc_logsoftmax(agg2, y2, dinv, b2p)
    return out[:N, :DOUT]


# speedup vs baseline: 3.4080x; 2.6040x over previous
import functools

import jax
import jax.numpy as jnp
from jax import lax
from jax.experimental import pallas as pl
from jax.experimental.pallas import tpu as pltpu
from jax.experimental.pallas import tpu_sc as plsc

N = 10000
E = 320000
DIN = 128
DH = 256
DOUT = 40
DOP = 128

NC = 2
NS = 16
NW = NC * NS

NPAD = 10240
EPAD = 327680
BLK_E = 128
NBLK = EPAD // (NW * BLK_E)
ZROWS = NPAD // NS
ZB = ZROWS // BLK_E

NBLK0 = 80
NBLK1 = 80
PNB = 16
assert NBLK0 + NBLK1 == 2 * NBLK
assert NBLK0 % PNB == 0 and NBLK1 % PNB == 0 and PNB % 2 == 0


def _sc_degree(dst2d):
    mesh = plsc.VectorSubcoreMesh(core_axis_name="c", subcore_axis_name="s")

    @functools.partial(
        pl.kernel,
        mesh=mesh,
        out_type=jax.ShapeDtypeStruct((NC, NPAD, 16), jnp.float32),
        compiler_params=pltpu.CompilerParams(use_tc_tiling_on_sc=False),
        scratch_types=[
            pltpu.VMEM((NBLK, BLK_E), jnp.int32),
            pltpu.VMEM((BLK_E, 16), jnp.float32),
            pltpu.VMEM((BLK_E, 16), jnp.float32),
            pltpu.VMEM_SHARED((NPAD, 16), jnp.float32),
        ],
    )
    def k(dst_hbm, out_hbm, dst_v, ones_v, zeros_v, acc):
        c = lax.axis_index("c")
        s = lax.axis_index("s")
        wid = c * NS + s

        def fill(i, carry):
            ones_v[i, :] = jnp.ones((16,), jnp.float32)
            zeros_v[i, :] = jnp.zeros((16,), jnp.float32)
            return carry

        lax.fori_loop(0, BLK_E, fill, 0)

        def zero(b, carry):
            pltpu.sync_copy(zeros_v, acc.at[pl.ds(s * ZROWS + b * BLK_E, BLK_E)])
            return carry

        lax.fori_loop(0, ZB, zero, 0)
        pltpu.sync_copy(dst_hbm.at[pl.ds(wid * NBLK, NBLK)], dst_v)
        plsc.subcore_barrier()

        def body(j, carry):
            pltpu.sync_copy(ones_v, acc.at[dst_v.at[j]], add=True)
            return carry

        lax.fori_loop(0, NBLK, body, 0)
        plsc.subcore_barrier()
        pltpu.sync_copy(
            acc.at[pl.ds(s * ZROWS, ZROWS)],
            out_hbm.at[c, pl.ds(s * ZROWS, ZROWS)],
        )

    return k(dst2d)


def _sc_agg(y, comb2d, d):
    mesh = plsc.VectorSubcoreMesh(core_axis_name="c", subcore_axis_name="s")

    @functools.partial(
        pl.kernel,
        mesh=mesh,
        out_type=jax.ShapeDtypeStruct((NC, NPAD, d), jnp.float32),
        compiler_params=pltpu.CompilerParams(use_tc_tiling_on_sc=False),
        scratch_types=[
            pltpu.VMEM((2 * PNB, BLK_E), jnp.int32),
            pltpu.VMEM((2 * PNB, BLK_E), jnp.int32),
            pltpu.VMEM((BLK_E, d), jnp.float32),
            pltpu.VMEM((BLK_E, d), jnp.float32),
            pltpu.VMEM_SHARED((NPAD, d), jnp.float32),
            pltpu.SemaphoreType.DMA,
            pltpu.SemaphoreType.DMA,
            pltpu.SemaphoreType.DMA,
            pltpu.SemaphoreType.DMA,
        ],
    )
    def k(y_hbm, comb_hbm, out_hbm, ib0, ib1, rows0, rows1, acc,
          sem0, sem1, semi0, semi1):
        c = lax.axis_index("c")
        s = lax.axis_index("s")
        base_blk = jnp.where(c == 0, s * NBLK0, NS * NBLK0 + s * NBLK1)
        nph = jnp.where(c == 0, NBLK0 // PNB, NBLK1 // PNB)

        def fetch_idx(p, buf, sem):
            pltpu.async_copy(
                comb_hbm.at[pl.ds(2 * (base_blk + p * PNB), 2 * PNB)], buf, sem)

        def drain_idx(buf, sem):
            pltpu.make_async_copy(comb_hbm.at[pl.ds(0, 2 * PNB)], buf,
                                  sem).wait()

        @pl.when(nph > 0)
        def _():
            fetch_idx(0, ib0, semi0)

        def zrow(i, carry):
            def zcol(j, carry2):
                rows0[i, pl.ds(j * 16, 16)] = jnp.zeros((16,), jnp.float32)
                return carry2

            lax.fori_loop(0, d // 16, zcol, 0)
            return carry

        lax.fori_loop(0, BLK_E, zrow, 0)

        def zero(b, carry):
            pltpu.sync_copy(rows0, acc.at[pl.ds(s * ZROWS + b * BLK_E, BLK_E)])
            return carry

        lax.fori_loop(0, ZB, zero, 0)
        plsc.subcore_barrier()

        def gather(ib, j, buf, sem):
            pltpu.async_copy(y_hbm.at[ib.at[2 * j]], buf, sem)

        def drain(buf, sem):
            pltpu.make_async_copy(y_hbm.at[pl.ds(0, BLK_E)], buf, sem).wait()

        def scatter(ib, j, buf):
            pltpu.sync_copy(buf, acc.at[ib.at[2 * j + 1]], add=True)

        def run_phase(ib):
            gather(ib, 0, rows0, sem0)

            def body(i, carry):
                j = 2 * i
                gather(ib, j + 1, rows1, sem1)
                drain(rows0, sem0)
                scatter(ib, j, rows0)
                gather(ib, j + 2, rows0, sem0)
                drain(rows1, sem1)
                scatter(ib, j + 1, rows1)
                return carry

            lax.fori_loop(0, PNB // 2 - 1, body, 0)
            gather(ib, PNB - 1, rows1, sem1)
            drain(rows0, sem0)
            scatter(ib, PNB - 2, rows0)
            drain(rows1, sem1)
            scatter(ib, PNB - 1, rows1)

        def phase(p, carry):
            @pl.when(p % 2 == 0)
            def _():
                @pl.when(p + 1 < nph)
                def _():
                    fetch_idx(p + 1, ib1, semi1)

                drain_idx(ib0, semi0)
                run_phase(ib0)

            @pl.when(p % 2 == 1)
            def _():
                @pl.when(p + 1 < nph)
                def _():
                    fetch_idx(p + 1, ib0, semi0)

                drain_idx(ib1, semi1)
                run_phase(ib1)

            return carry

        lax.fori_loop(0, nph, phase, 0)
        plsc.subcore_barrier()
        pltpu.sync_copy(
            acc.at[pl.ds(s * ZROWS, ZROWS)],
            out_hbm.at[c, pl.ds(s * ZROWS, ZROWS)],
        )

    return k(y, comb2d)


def _tc_prep(deg_parts, x_pad):
    blk = 1024

    def body(d_ref, x_ref, dinv_ref, y_ref):
        deg = d_ref[0, :, 0:1] + d_ref[1, :, 0:1] + 1.0
        dinv = lax.rsqrt(deg)
        dinv_ref[...] = dinv
        y_ref[...] = x_ref[...] * dinv

    return pl.pallas_call(
        body,
        grid=(NPAD // blk,),
        in_specs=[
            pl.BlockSpec((NC, blk, 16), lambda i: (0, i, 0)),
            pl.BlockSpec((blk, DIN), lambda i: (i, 0)),
        ],
        out_specs=[
            pl.BlockSpec((blk, 1), lambda i: (i, 0)),
            pl.BlockSpec((blk, DIN), lambda i: (i, 0)),
        ],
        out_shape=[
            jax.ShapeDtypeStruct((NPAD, 1), jnp.float32),
            jax.ShapeDtypeStruct((NPAD, DIN), jnp.float32),
        ],
    )(deg_parts, x_pad)


def _tc_mlp(agg_parts, y, dinv, w1, b1r, w2p):
    blk = 512

    def body(a_ref, y_ref, dinv_ref, w1_ref, b1_ref, w2_ref, out_ref):
        z = (a_ref[0] + a_ref[1] + y_ref[...]) * dinv_ref[...]
        h = jnp.dot(z, w1_ref[...], preferred_element_type=jnp.float32)
        h = jnp.maximum(h + b1_ref[...], 0.0)
        p = jnp.dot(h, w2_ref[...], preferred_element_type=jnp.float32)
        out_ref[...] = p * dinv_ref[...]

    return pl.pallas_call(
        body,
        grid=(NPAD // blk,),
        in_specs=[
            pl.BlockSpec((NC, blk, DIN), lambda i: (0, i, 0)),
            pl.BlockSpec((blk, DIN), lambda i: (i, 0)),
            pl.BlockSpec((blk, 1), lambda i: (i, 0)),
            pl.BlockSpec((DIN, DH), lambda i: (0, 0)),
            pl.BlockSpec((1, DH), lambda i: (0, 0)),
            pl.BlockSpec((DH, DOP), lambda i: (0, 0)),
        ],
        out_specs=pl.BlockSpec((blk, DOP), lambda i: (i, 0)),
        out_shape=jax.ShapeDtypeStruct((NPAD, DOP), jnp.float32),
    )(agg_parts, y, dinv, w1, b1r, w2p)


def _tc_logsoftmax(agg_parts, y2, dinv, b2p):
    blk = 1024

    def body(a_ref, y2_ref, dinv_ref, b2_ref, out_ref):
        z = (a_ref[0] + a_ref[1] + y2_ref[...]) * dinv_ref[...] + b2_ref[...]
        col = lax.broadcasted_iota(jnp.int32, z.shape, 1)
        mask = col < DOUT
        zm = jnp.where(mask, z, -jnp.inf)
        m = jnp.max(zm, axis=1, keepdims=True)
        e = jnp.where(mask, jnp.exp(z - m), 0.0)
        lse = jnp.log(jnp.sum(e, axis=1, keepdims=True))
        out_ref[...] = z - m - lse

    return pl.pallas_call(
        body,
        grid=(NPAD // blk,),
        in_specs=[
            pl.BlockSpec((NC, blk, DOP), lambda i: (0, i, 0)),
            pl.BlockSpec((blk, DOP), lambda i: (i, 0)),
            pl.BlockSpec((blk, 1), lambda i: (i, 0)),
            pl.BlockSpec((1, DOP), lambda i: (0, 0)),
        ],
        out_specs=pl.BlockSpec((blk, DOP), lambda i: (i, 0)),
        out_shape=jax.ShapeDtypeStruct((NPAD, DOP), jnp.float32),
    )(agg_parts, y2, dinv, b2p)


def kernel(x, edge_index, W1, b1, W2, b2):
    src = edge_index[0]
    dst = edge_index[1]
    pad = EPAD - E
    pad_ids = jnp.arange(pad, dtype=jnp.int32)
    src2d = jnp.concatenate(
        [src, pad_ids % N]).reshape(EPAD // BLK_E, BLK_E)
    dst2d = jnp.concatenate(
        [dst, N + pad_ids % (NPAD - N)]).reshape(EPAD // BLK_E, BLK_E)
    comb2d = jnp.stack([src2d, dst2d], axis=1).reshape(2 * EPAD // BLK_E, BLK_E)
    x_pad = jnp.pad(x, ((0, NPAD - N), (0, 0)))
    w2p = jnp.pad(W2, ((0, 0), (0, DOP - DOUT)))
    b1r = b1.reshape(1, DH)
    b2p = jnp.pad(b2, (0, DOP - DOUT)).reshape(1, DOP)

    deg_parts = _sc_degree(dst2d)
    dinv, y = _tc_prep(deg_parts, x_pad)
    agg1 = _sc_agg(y, comb2d, DIN)
    y2 = _tc_mlp(agg1, y, dinv, W1, b1r, w2p)
    agg2 = _sc_agg(y2, comb2d, DOP)
    out = _tc_logsoftmax(agg2, y2, dinv, b2p)
    return out[:N, :DOUT]


# speedup vs baseline: 3.7238x; 1.0926x over previous
import functools

import jax
import jax.numpy as jnp
from jax import lax
from jax.experimental import pallas as pl
from jax.experimental.pallas import tpu as pltpu
from jax.experimental.pallas import tpu_sc as plsc

N = 10000
E = 320000
DIN = 128
DH = 256
DOUT = 40
DOP = 64

NC = 2
NS = 16
NW = NC * NS

NPAD = 10240
EPAD = 327680
BLK_E = 128
NBLK = EPAD // (NW * BLK_E)
ZROWS = NPAD // NS
ZB = ZROWS // BLK_E

NBLK0 = 80
NBLK1 = 80
PNB = 16
assert NBLK0 + NBLK1 == 2 * NBLK
assert NBLK0 % PNB == 0 and NBLK1 % PNB == 0 and PNB % 2 == 0


def _sc_degree(dst2d):
    mesh = plsc.VectorSubcoreMesh(core_axis_name="c", subcore_axis_name="s")

    @functools.partial(
        pl.kernel,
        mesh=mesh,
        out_type=jax.ShapeDtypeStruct((NC, NPAD, 16), jnp.float32),
        compiler_params=pltpu.CompilerParams(use_tc_tiling_on_sc=False),
        scratch_types=[
            pltpu.VMEM((NBLK, BLK_E), jnp.int32),
            pltpu.VMEM((BLK_E, 16), jnp.float32),
            pltpu.VMEM((BLK_E, 16), jnp.float32),
            pltpu.VMEM_SHARED((NPAD, 16), jnp.float32),
        ],
    )
    def k(dst_hbm, out_hbm, dst_v, ones_v, zeros_v, acc):
        c = lax.axis_index("c")
        s = lax.axis_index("s")
        wid = c * NS + s

        def fill(i, carry):
            ones_v[i, :] = jnp.ones((16,), jnp.float32)
            zeros_v[i, :] = jnp.zeros((16,), jnp.float32)
            return carry

        lax.fori_loop(0, BLK_E, fill, 0)

        def zero(b, carry):
            pltpu.sync_copy(zeros_v, acc.at[pl.ds(s * ZROWS + b * BLK_E, BLK_E)])
            return carry

        lax.fori_loop(0, ZB, zero, 0)
        pltpu.sync_copy(dst_hbm.at[pl.ds(wid * NBLK, NBLK)], dst_v)
        plsc.subcore_barrier()

        def body(j, carry):
            pltpu.sync_copy(ones_v, acc.at[dst_v.at[j]], add=True)
            return carry

        lax.fori_loop(0, NBLK, body, 0)
        plsc.subcore_barrier()
        pltpu.sync_copy(
            acc.at[pl.ds(s * ZROWS, ZROWS)],
            out_hbm.at[c, pl.ds(s * ZROWS, ZROWS)],
        )

    return k(dst2d)


def _sc_agg(y, comb2d, d):
    mesh = plsc.VectorSubcoreMesh(core_axis_name="c", subcore_axis_name="s")

    @functools.partial(
        pl.kernel,
        mesh=mesh,
        out_type=jax.ShapeDtypeStruct((NC, NPAD, d), jnp.float32),
        compiler_params=pltpu.CompilerParams(use_tc_tiling_on_sc=False),
        scratch_types=[
            pltpu.VMEM((2 * PNB, BLK_E), jnp.int32),
            pltpu.VMEM((2 * PNB, BLK_E), jnp.int32),
            pltpu.VMEM((BLK_E, d), jnp.float32),
            pltpu.VMEM((BLK_E, d), jnp.float32),
            pltpu.VMEM_SHARED((NPAD, d), jnp.float32),
            pltpu.SemaphoreType.DMA,
            pltpu.SemaphoreType.DMA,
            pltpu.SemaphoreType.DMA,
            pltpu.SemaphoreType.DMA,
        ],
    )
    def k(y_hbm, comb_hbm, out_hbm, ib0, ib1, rows0, rows1, acc,
          sem0, sem1, semi0, semi1):
        c = lax.axis_index("c")
        s = lax.axis_index("s")
        base_blk = jnp.where(c == 0, s * NBLK0, NS * NBLK0 + s * NBLK1)
        nph = jnp.where(c == 0, NBLK0 // PNB, NBLK1 // PNB)

        def fetch_idx(p, buf, sem):
            pltpu.async_copy(
                comb_hbm.at[pl.ds(2 * (base_blk + p * PNB), 2 * PNB)], buf, sem)

        def drain_idx(buf, sem):
            pltpu.make_async_copy(comb_hbm.at[pl.ds(0, 2 * PNB)], buf,
                                  sem).wait()

        @pl.when(nph > 0)
        def _():
            fetch_idx(0, ib0, semi0)

        def zrow(i, carry):
            def zcol(j, carry2):
                rows0[i, pl.ds(j * 16, 16)] = jnp.zeros((16,), jnp.float32)
                return carry2

            lax.fori_loop(0, d // 16, zcol, 0)
            return carry

        lax.fori_loop(0, BLK_E, zrow, 0)

        def zero(b, carry):
            pltpu.sync_copy(rows0, acc.at[pl.ds(s * ZROWS + b * BLK_E, BLK_E)])
            return carry

        lax.fori_loop(0, ZB, zero, 0)
        plsc.subcore_barrier()

        def gather(ib, j, buf, sem):
            pltpu.async_copy(y_hbm.at[ib.at[2 * j]], buf, sem)

        def drain(buf, sem):
            pltpu.make_async_copy(y_hbm.at[pl.ds(0, BLK_E)], buf, sem).wait()

        def scatter(ib, j, buf):
            pltpu.sync_copy(buf, acc.at[ib.at[2 * j + 1]], add=True)

        def run_phase(ib):
            gather(ib, 0, rows0, sem0)

            def body(i, carry):
                j = 2 * i
                gather(ib, j + 1, rows1, sem1)
                drain(rows0, sem0)
                scatter(ib, j, rows0)
                gather(ib, j + 2, rows0, sem0)
                drain(rows1, sem1)
                scatter(ib, j + 1, rows1)
                return carry

            lax.fori_loop(0, PNB // 2 - 1, body, 0)
            gather(ib, PNB - 1, rows1, sem1)
            drain(rows0, sem0)
            scatter(ib, PNB - 2, rows0)
            drain(rows1, sem1)
            scatter(ib, PNB - 1, rows1)

        def phase(p, carry):
            @pl.when(p % 2 == 0)
            def _():
                @pl.when(p + 1 < nph)
                def _():
                    fetch_idx(p + 1, ib1, semi1)

                drain_idx(ib0, semi0)
                run_phase(ib0)

            @pl.when(p % 2 == 1)
            def _():
                @pl.when(p + 1 < nph)
                def _():
                    fetch_idx(p + 1, ib0, semi0)

                drain_idx(ib1, semi1)
                run_phase(ib1)

            return carry

        lax.fori_loop(0, nph, phase, 0)
        plsc.subcore_barrier()
        pltpu.sync_copy(
            acc.at[pl.ds(s * ZROWS, ZROWS)],
            out_hbm.at[c, pl.ds(s * ZROWS, ZROWS)],
        )

    return k(y, comb2d)


def _tc_prep(deg_parts, x_pad):
    blk = 1024

    def body(d_ref, x_ref, dinv_ref, y_ref):
        deg = d_ref[0, :, 0:1] + d_ref[1, :, 0:1] + 1.0
        dinv = lax.rsqrt(deg)
        dinv_ref[...] = dinv
        y_ref[...] = x_ref[...] * dinv

    return pl.pallas_call(
        body,
        grid=(NPAD // blk,),
        in_specs=[
            pl.BlockSpec((NC, blk, 16), lambda i: (0, i, 0)),
            pl.BlockSpec((blk, DIN), lambda i: (i, 0)),
        ],
        out_specs=[
            pl.BlockSpec((blk, 1), lambda i: (i, 0)),
            pl.BlockSpec((blk, DIN), lambda i: (i, 0)),
        ],
        out_shape=[
            jax.ShapeDtypeStruct((NPAD, 1), jnp.float32),
            jax.ShapeDtypeStruct((NPAD, DIN), jnp.float32),
        ],
    )(deg_parts, x_pad)


def _tc_mlp(agg_parts, y, dinv, w1, b1r, w2p):
    blk = 512

    def body(a_ref, y_ref, dinv_ref, w1_ref, b1_ref, w2_ref, out_ref):
        z = (a_ref[0] + a_ref[1] + y_ref[...]) * dinv_ref[...]
        h = jnp.dot(z, w1_ref[...], preferred_element_type=jnp.float32)
        h = jnp.maximum(h + b1_ref[...], 0.0)
        p = jnp.dot(h, w2_ref[...], preferred_element_type=jnp.float32)
        out_ref[...] = p * dinv_ref[...]

    return pl.pallas_call(
        body,
        grid=(NPAD // blk,),
        in_specs=[
            pl.BlockSpec((NC, blk, DIN), lambda i: (0, i, 0)),
            pl.BlockSpec((blk, DIN), lambda i: (i, 0)),
            pl.BlockSpec((blk, 1), lambda i: (i, 0)),
            pl.BlockSpec((DIN, DH), lambda i: (0, 0)),
            pl.BlockSpec((1, DH), lambda i: (0, 0)),
            pl.BlockSpec((DH, DOP), lambda i: (0, 0)),
        ],
        out_specs=pl.BlockSpec((blk, DOP), lambda i: (i, 0)),
        out_shape=jax.ShapeDtypeStruct((NPAD, DOP), jnp.float32),
    )(agg_parts, y, dinv, w1, b1r, w2p)


def _tc_logsoftmax(agg_parts, y2, dinv, b2p):
    blk = 1024

    def body(a_ref, y2_ref, dinv_ref, b2_ref, out_ref):
        z = (a_ref[0] + a_ref[1] + y2_ref[...]) * dinv_ref[...] + b2_ref[...]
        col = lax.broadcasted_iota(jnp.int32, z.shape, 1)
        mask = col < DOUT
        zm = jnp.where(mask, z, -jnp.inf)
        m = jnp.max(zm, axis=1, keepdims=True)
        e = jnp.where(mask, jnp.exp(z - m), 0.0)
        lse = jnp.log(jnp.sum(e, axis=1, keepdims=True))
        out_ref[...] = z - m - lse

    return pl.pallas_call(
        body,
        grid=(NPAD // blk,),
        in_specs=[
            pl.BlockSpec((NC, blk, DOP), lambda i: (0, i, 0)),
            pl.BlockSpec((blk, DOP), lambda i: (i, 0)),
            pl.BlockSpec((blk, 1), lambda i: (i, 0)),
            pl.BlockSpec((1, DOP), lambda i: (0, 0)),
        ],
        out_specs=pl.BlockSpec((blk, DOP), lambda i: (i, 0)),
        out_shape=jax.ShapeDtypeStruct((NPAD, DOP), jnp.float32),
    )(agg_parts, y2, dinv, b2p)


def kernel(x, edge_index, W1, b1, W2, b2):
    src = edge_index[0]
    dst = edge_index[1]
    pad = EPAD - E
    pad_ids = jnp.arange(pad, dtype=jnp.int32)
    src2d = jnp.concatenate(
        [src, pad_ids % N]).reshape(EPAD // BLK_E, BLK_E)
    dst2d = jnp.concatenate(
        [dst, N + pad_ids % (NPAD - N)]).reshape(EPAD // BLK_E, BLK_E)
    comb2d = jnp.stack([src2d, dst2d], axis=1).reshape(2 * EPAD // BLK_E, BLK_E)
    x_pad = jnp.pad(x, ((0, NPAD - N), (0, 0)))
    w2p = jnp.pad(W2, ((0, 0), (0, DOP - DOUT)))
    b1r = b1.reshape(1, DH)
    b2p = jnp.pad(b2, (0, DOP - DOUT)).reshape(1, DOP)

    deg_parts = _sc_degree(dst2d)
    dinv, y = _tc_prep(deg_parts, x_pad)
    agg1 = _sc_agg(y, comb2d, DIN)
    y2 = _tc_mlp(agg1, y, dinv, W1, b1r, w2p)
    agg2 = _sc_agg(y2, comb2d, DOP)
    out = _tc_logsoftmax(agg2, y2, dinv, b2p)
    return out[:N, :DOUT]
